# Initial kernel scaffold; baseline (speedup 1.0000x reference)
import jax
import jax.numpy as jnp
from jax.experimental import pallas as pl


def kernel(x, edge_index, W1, b1, W2, b2, W3, b3):
    raise NotImplementedError("write your pallas kernel here")


# speedup vs baseline: 15.8065x; 15.8065x over previous
import functools

import jax
import jax.numpy as jnp
from jax import lax
from jax.experimental import pallas as pl
from jax.experimental.pallas import tpu as pltpu
from jax.experimental.pallas import tpu_sc as plsc

NC = 2
NS = 16
C = 80


def _mesh():
  return plsc.VectorSubcoreMesh(core_axis_name="c", subcore_axis_name="s")


def _fill_2d(ref, nrows, value):
  def outer(i, _):
    def inner(j, _):
      ref[i, pl.ds(j * 16, 16)] = jnp.full((16,), value, jnp.float32)
      return 0
    lax.fori_loop(0, 8, inner, 0)
    return 0
  lax.fori_loop(0, nrows, outer, 0)


def _sc_scatter(src3, dst3, g2d, n, with_gather):
  nt, iters, c_ = src3.shape
  rps = (n // NS) // 8 * 8
  tail = n - NS * rps
  zr = 16
  assert rps % zr == 0 and tail <= zr

  @functools.partial(
      pl.kernel,
      out_type=jax.ShapeDtypeStruct((NC, n, 128), jnp.float32),
      mesh=_mesh(),
      scratch_types=[
          pltpu.VMEM_SHARED((n, 128), jnp.float32),
          pltpu.VMEM((iters, C), jnp.int32),
          pltpu.VMEM((iters, C), jnp.int32),
          pltpu.VMEM((C, 128), jnp.float32),
          pltpu.VMEM((zr, 128), jnp.float32),
          pltpu.SemaphoreType.DMA,
      ],
  )
  def k(src_hbm, dst_hbm, g_hbm, out_hbm, agg_sh, sidx, didx, rows_v, zbuf,
        sem):
    c = lax.axis_index("c")
    s = lax.axis_index("s")
    t = c * NS + s

    _fill_2d(zbuf, zr, 0.0)
    if not with_gather:
      _fill_2d(rows_v, C, 1.0)

    def zcopy(kk, _):
      pltpu.sync_copy(zbuf, agg_sh.at[pl.ds(s * rps + kk * zr, zr)])
      return 0
    lax.fori_loop(0, rps // zr, zcopy, 0)

    @pl.when(s == 0)
    def _():
      pltpu.sync_copy(zbuf.at[pl.ds(0, tail)], agg_sh.at[pl.ds(NS * rps, tail)])

    if with_gather:
      pltpu.sync_copy(src_hbm.at[t], sidx)
    pltpu.sync_copy(dst_hbm.at[t], didx)
    plsc.subcore_barrier()

    def body(i, _):
      if with_gather:
        pltpu.async_copy(g_hbm.at[sidx.at[i]], rows_v, sem).wait()
      pltpu.sync_copy(rows_v, agg_sh.at[didx.at[i]], add=True)
      return 0
    lax.fori_loop(0, iters, body, 0)

    plsc.subcore_barrier()
    pltpu.sync_copy(agg_sh.at[pl.ds(s * rps, rps)],
                    out_hbm.at[c, pl.ds(s * rps, rps)])

    @pl.when(s == 0)
    def _():
      pltpu.sync_copy(agg_sh.at[pl.ds(NS * rps, tail)],
                      out_hbm.at[c, pl.ds(NS * rps, tail)])

  return k(src3, dst3, g2d)


def _dis_block(degp_ref):
  deg = degp_ref[0][:, :1] + degp_ref[1][:, :1] + 1.0
  return lax.rsqrt(deg)


def _tc_first(x, degp, w, n, r=1000):
  def body(x_ref, degp_ref, w_ref, o_ref):
    dis = _dis_block(degp_ref)
    o_ref[...] = jnp.dot(x_ref[...] * dis, w_ref[...],
                         preferred_element_type=jnp.float32)

  return pl.pallas_call(
      body,
      grid=(n // r,),
      in_specs=[
          pl.BlockSpec((r, 128), lambda i: (i, 0)),
          pl.BlockSpec((NC, r, 128), lambda i: (0, i, 0)),
          pl.BlockSpec((128, 128), lambda i: (0, 0)),
      ],
      out_specs=pl.BlockSpec((r, 128), lambda i: (i, 0)),
      out_shape=jax.ShapeDtypeStruct((n, 128), jnp.float32),
  )(x, degp, w)


def _tc_mid(aggp, g, degp, b, w, n, r=1000):
  def body(aggp_ref, g_ref, degp_ref, b_ref, w_ref, o_ref):
    dis = _dis_block(degp_ref)
    agg = aggp_ref[0] + aggp_ref[1] + g_ref[...]
    h = jnp.maximum(agg * dis + b_ref[...], 0.0)
    o_ref[...] = jnp.dot(h * dis, w_ref[...],
                         preferred_element_type=jnp.float32)

  return pl.pallas_call(
      body,
      grid=(n // r,),
      in_specs=[
          pl.BlockSpec((NC, r, 128), lambda i: (0, i, 0)),
          pl.BlockSpec((r, 128), lambda i: (i, 0)),
          pl.BlockSpec((NC, r, 128), lambda i: (0, i, 0)),
          pl.BlockSpec((1, 128), lambda i: (0, 0)),
          pl.BlockSpec((128, 128), lambda i: (0, 0)),
      ],
      out_specs=pl.BlockSpec((r, 128), lambda i: (i, 0)),
      out_shape=jax.ShapeDtypeStruct((n, 128), jnp.float32),
  )(aggp, g, degp, b, w)


def _tc_final(aggp, g, degp, b, n, r=1000):
  def body(aggp_ref, g_ref, degp_ref, b_ref, o_ref):
    dis = _dis_block(degp_ref)
    agg = aggp_ref[0] + aggp_ref[1] + g_ref[...]
    o_ref[...] = agg * dis + b_ref[...]

  return pl.pallas_call(
      body,
      grid=(n // r,),
      in_specs=[
          pl.BlockSpec((NC, r, 128), lambda i: (0, i, 0)),
          pl.BlockSpec((r, 128), lambda i: (i, 0)),
          pl.BlockSpec((NC, r, 128), lambda i: (0, i, 0)),
          pl.BlockSpec((1, 128), lambda i: (0, 0)),
      ],
      out_specs=pl.BlockSpec((r, 128), lambda i: (i, 0)),
      out_shape=jax.ShapeDtypeStruct((n, 128), jnp.float32),
  )(aggp, g, degp, b)


def kernel(x, edge_index, W1, b1, W2, b2, W3, b3):
  n, d = x.shape
  e = edge_index.shape[1]
  nt = NC * NS
  assert e % (nt * C) == 0 and d == 128
  iters = e // (nt * C)

  src3 = edge_index[0].reshape(nt, iters, C)
  dst3 = edge_index[1].reshape(nt, iters, C)
  b1r = b1.reshape(1, 128)
  b2r = b2.reshape(1, 128)
  b3r = b3.reshape(1, 128)

  degp = _sc_scatter(src3, dst3, x, n, with_gather=False)

  g1 = _tc_first(x, degp, W1, n)
  a1 = _sc_scatter(src3, dst3, g1, n, with_gather=True)
  g2 = _tc_mid(a1, g1, degp, b1r, W2, n)
  a2 = _sc_scatter(src3, dst3, g2, n, with_gather=True)
  g3 = _tc_mid(a2, g2, degp, b2r, W3, n)
  a3 = _sc_scatter(src3, dst3, g3, n, with_gather=True)
  return _tc_final(a3, g3, degp, b3r, n)
